# baseline (speedup 1.0000x reference)
import functools

import jax
import jax.numpy as jnp
from jax import lax
from jax.experimental import pallas as pl
from jax.experimental.pallas import tpu as pltpu
from jax.experimental.pallas import tpu_sc as plsc

N_NODES = 100000
IN_CH = 768
HID = 64
NT = 1024
N_EDGES = 3200000
ALPHA = 0.0001

NC = 2
NS = 16
NW = NC * NS
EDGES_PER_TILE = N_EDGES // NW
CHUNK = 2000
NGROUP = CHUNK // 16
NCHUNK = EDGES_PER_TILE // CHUNK
K = 128
TRASH = NT
ACC_ROWS = NT + 8
TGT_PER_TILE = NT // NW
ROW_BLK = 2000
NOT_TGT = 0xFFFF


def _proj_body(x_ref, w_ref, y_ref):
    y_ref[...] = jnp.dot(x_ref[...], w_ref[...],
                         preferred_element_type=jnp.float32)


def _project(x, wt):
    return pl.pallas_call(
        _proj_body,
        grid=(N_NODES // ROW_BLK,),
        in_specs=[
            pl.BlockSpec((ROW_BLK, IN_CH), lambda i: (i, 0)),
            pl.BlockSpec((IN_CH, HID), lambda i: (0, 0)),
        ],
        out_specs=pl.BlockSpec((ROW_BLK, HID), lambda i: (i, 0)),
        out_shape=jax.ShapeDtypeStruct((N_NODES, HID), jnp.float32),
    )(x, wt)


def _sc_body(src_hbm, dst_hbm, lut_hbm, y_hbm, tgt_hbm, z64_hbm, z16_hbm,
             ones_hbm, acc_out, cnt_out, yt_out,
             lut_v, srcc_v, dstc_v, sbuf_v, slbuf_v, rows_v, ones_v,
             ti_v, tr_v, acc_sh, cnt_sh):
    cid = lax.axis_index("c")
    sid = lax.axis_index("s")
    wid = sid * NC + cid

    @pl.when(sid == 0)
    def _():
        pltpu.sync_copy(z64_hbm, acc_sh)
        pltpu.sync_copy(z16_hbm, cnt_sh)
    plsc.subcore_barrier()

    pltpu.sync_copy(lut_hbm, lut_v)
    pltpu.sync_copy(ones_hbm, ones_v)

    zeros16 = jnp.zeros((16,), jnp.int32)
    trash16 = jnp.full((16,), TRASH, jnp.int32)
    for i in range(K // 16):
        sbuf_v[pl.ds(16 * i, 16)] = zeros16
        slbuf_v[pl.ds(16 * i, 16)] = trash16

    def flush(_):
        pltpu.sync_copy(y_hbm.at[sbuf_v], rows_v)
        pltpu.sync_copy(rows_v, acc_sh.at[slbuf_v], add=True)
        pltpu.sync_copy(ones_v, cnt_sh.at[slbuf_v], add=True)
        for i in range(K // 16):
            slbuf_v[pl.ds(16 * i, 16)] = trash16
        return 0

    ebase = wid * EDGES_PER_TILE

    def chunk_body(c, cur0):
        base = ebase + c * CHUNK
        pltpu.sync_copy(src_hbm.at[pl.ds(base, CHUNK)], srcc_v)
        pltpu.sync_copy(dst_hbm.at[pl.ds(base, CHUNK)], dstc_v)

        def grp(g, cur):
            d16 = dstc_v[pl.ds(g * 16, 16)]
            w16 = plsc.load_gather(lut_v, [lax.shift_right_logical(d16, 1)])
            shmt = (d16 & 1) * 16
            half = lax.shift_right_logical(w16, shmt) & NOT_TGT
            m = half != NOT_TGT

            def hit(cur):
                cur = lax.cond(cur > K - 16, flush, lambda c: c, cur)
                s16 = srcc_v[pl.ds(g * 16, 16)]
                plsc.store_compressed(sbuf_v.at[pl.ds(cur, 16)], s16, mask=m)
                plsc.store_compressed(slbuf_v.at[pl.ds(cur, 16)], half,
                                      mask=m)
                return cur + jnp.sum(m.astype(jnp.int32))

            return lax.cond(jnp.any(m), hit, lambda c: c, cur)

        return lax.fori_loop(0, NGROUP, grp, cur0)

    lax.fori_loop(0, NCHUNK, chunk_body, 0)
    flush(0)

    plsc.subcore_barrier()

    @pl.when(sid == 0)
    def _():
        pltpu.sync_copy(acc_sh, acc_out.at[cid])
        pltpu.sync_copy(cnt_sh, cnt_out.at[cid])

    tbase = wid * TGT_PER_TILE
    pltpu.sync_copy(tgt_hbm.at[pl.ds(tbase, TGT_PER_TILE)], ti_v)
    pltpu.sync_copy(y_hbm.at[ti_v], tr_v)
    pltpu.sync_copy(tr_v, yt_out.at[pl.ds(tbase, TGT_PER_TILE)])


@functools.cache
def _make_sc_aggregate():
    return pl.kernel(
        _sc_body,
        out_type=(
            jax.ShapeDtypeStruct((NC, ACC_ROWS, HID), jnp.float32),
            jax.ShapeDtypeStruct((NC, ACC_ROWS, 16), jnp.float32),
            jax.ShapeDtypeStruct((NT, HID), jnp.float32),
        ),
        mesh=plsc.VectorSubcoreMesh(core_axis_name="c", subcore_axis_name="s"),
        compiler_params=pltpu.CompilerParams(needs_layout_passes=False,
                                             use_tc_tiling_on_sc=False),
        scratch_types=[
            pltpu.VMEM((N_NODES // 2,), jnp.int32),
            pltpu.VMEM((CHUNK,), jnp.int32),
            pltpu.VMEM((CHUNK,), jnp.int32),
            pltpu.VMEM((K,), jnp.int32),
            pltpu.VMEM((K,), jnp.int32),
            pltpu.VMEM((K, HID), jnp.float32),
            pltpu.VMEM((K, 16), jnp.float32),
            pltpu.VMEM((TGT_PER_TILE,), jnp.int32),
            pltpu.VMEM((TGT_PER_TILE, HID), jnp.float32),
            pltpu.VMEM_SHARED((ACC_ROWS, HID), jnp.float32),
            pltpu.VMEM_SHARED((ACC_ROWS, 16), jnp.float32),
        ],
    )


def _head_body(acc_ref, cnt_ref, sm_ref, yt_ref, wh_ref, bh_ref, bc_ref,
               o_ref):
    acc = acc_ref[0] + acc_ref[1]
    cntc = cnt_ref[0] + cnt_ref[1]
    cnt = cntc[:, 0:1]
    sm = sm_ref[...]
    iota_s = lax.broadcasted_iota(jnp.int32, (NT, ACC_ROWS), 1)
    p = (iota_s == sm).astype(jnp.float32)
    acc_t = jnp.dot(p, acc, preferred_element_type=jnp.float32)
    cnt_t = jnp.dot(p, cnt, preferred_element_type=jnp.float32)
    mean_t = acc_t * ALPHA / jnp.maximum(cnt_t, 1.0)
    h = jax.nn.relu(mean_t + yt_ref[...] + bc_ref[...])
    z = jnp.dot(h, wh_ref[...], preferred_element_type=jnp.float32)
    z = z + bh_ref[...]
    col = lax.broadcasted_iota(jnp.int32, (NT, 8), 1)
    o_ref[...] = jnp.where(col < 3, jax.nn.sigmoid(z), z)


def _heads(acc, cnt, slot_map, yt, wh, bh, bc):
    return pl.pallas_call(
        _head_body,
        out_shape=jax.ShapeDtypeStruct((NT, 8), jnp.float32),
    )(acc, cnt, slot_map, yt, wh, bh, bc)


@jax.jit
def kernel(x, edge_index, target_node_index, W_conv, b_conv, W_mu, b_mu,
           W_c, b_c):
    src = edge_index[0].astype(jnp.int32)
    dst = edge_index[1].astype(jnp.int32)
    tgt = target_node_index.astype(jnp.int32)

    y = _project(x, W_conv.T)

    lut = jnp.full((N_NODES,), -1, jnp.int32)
    lut = lut.at[tgt].set(jnp.arange(NT, dtype=jnp.int32))
    slot_map = lut[tgt].reshape(NT, 1)
    lutu = jnp.where(lut < 0, NOT_TGT, lut).astype(jnp.uint32)
    lut_packed = lax.bitcast_convert_type(
        lutu[0::2] | (lutu[1::2] << jnp.uint32(16)), jnp.int32)

    z64 = jnp.zeros((ACC_ROWS, HID), jnp.float32)
    z16 = jnp.zeros((ACC_ROWS, 16), jnp.float32)
    ones = jnp.ones((K, 16), jnp.float32)

    acc, cnt, yt = _make_sc_aggregate()(src, dst, lut_packed, y, tgt, z64,
                                        z16, ones)

    wh = jnp.zeros((8, HID), jnp.float32)
    wh = wh.at[0:3].set(W_mu).at[3].set(W_c[0])
    bh = jnp.zeros((1, 8), jnp.float32)
    bh = bh.at[0, 0:3].set(b_mu).at[0, 3].set(b_c[0])

    o = _heads(acc, cnt, slot_map, yt, wh.T, bh, b_conv.reshape(1, HID))

    mu = o[:, 0:3]
    state_value = o[:, 3:4]
    std = jnp.asarray(1e-05, dtype=jnp.float32)
    return (mu, std, state_value)


# speedup vs baseline: 1.0624x; 1.0624x over previous
import functools

import jax
import jax.numpy as jnp
from jax import lax
from jax.experimental import pallas as pl
from jax.experimental.pallas import tpu as pltpu
from jax.experimental.pallas import tpu_sc as plsc

N_NODES = 100000
IN_CH = 768
HID = 64
NT = 1024
N_EDGES = 3200000
ALPHA = 0.0001

NC = 2
NS = 16
NW = NC * NS
EDGES_PER_TILE = N_EDGES // NW
CHUNK = 2000
NGROUP = CHUNK // 16
NCHUNK = EDGES_PER_TILE // CHUNK
K = 128
TRASH = NT
ACC_ROWS = NT + 8
TGT_PER_TILE = NT // NW
ROW_BLK = 2000
NOT_TGT = 0xFFFF


def _proj_body(x_ref, w_ref, y_ref):
    y_ref[...] = jnp.dot(x_ref[...], w_ref[...],
                         preferred_element_type=jnp.float32)


def _project(x, wt):
    return pl.pallas_call(
        _proj_body,
        grid=(N_NODES // ROW_BLK,),
        in_specs=[
            pl.BlockSpec((ROW_BLK, IN_CH), lambda i: (i, 0)),
            pl.BlockSpec((IN_CH, HID), lambda i: (0, 0)),
        ],
        out_specs=pl.BlockSpec((ROW_BLK, HID), lambda i: (i, 0)),
        out_shape=jax.ShapeDtypeStruct((N_NODES, HID), jnp.float32),
    )(x, wt)


def _sc_body(src_hbm, dst_hbm, lut_hbm, y_hbm, tgt_hbm, z64_hbm, z16_hbm,
             ones_hbm, acc_out, cnt_out, yt_out,
             lut_v, srcc_v, dstc_v, half_buf, sbuf_v, slbuf_v, rows_v,
             ones_v, ti_v, tr_v, acc_sh, cnt_sh):
    cid = lax.axis_index("c")
    sid = lax.axis_index("s")
    wid = sid * NC + cid

    @pl.when(sid == 0)
    def _():
        pltpu.sync_copy(z64_hbm, acc_sh)
        pltpu.sync_copy(z16_hbm, cnt_sh)
    plsc.subcore_barrier()

    pltpu.sync_copy(lut_hbm, lut_v)
    pltpu.sync_copy(ones_hbm, ones_v)

    zeros16 = jnp.zeros((16,), jnp.int32)
    trash16 = jnp.full((16,), TRASH, jnp.int32)
    for i in range(K // 16):
        sbuf_v[pl.ds(16 * i, 16)] = zeros16
        slbuf_v[pl.ds(16 * i, 16)] = trash16

    def flush(_):
        pltpu.sync_copy(y_hbm.at[sbuf_v], rows_v)
        pltpu.sync_copy(rows_v, acc_sh.at[slbuf_v], add=True)
        pltpu.sync_copy(ones_v, cnt_sh.at[slbuf_v], add=True)
        for i in range(K // 16):
            slbuf_v[pl.ds(16 * i, 16)] = trash16
        return 0

    ebase = wid * EDGES_PER_TILE

    def chunk_body(c, cur0):
        base = ebase + c * CHUNK
        pltpu.sync_copy(src_hbm.at[pl.ds(base, CHUNK)], srcc_v)
        pltpu.sync_copy(dst_hbm.at[pl.ds(base, CHUNK)], dstc_v)

        @plsc.parallel_loop(0, NGROUP, unroll=4)
        def _(g):
            d16 = dstc_v[pl.ds(g * 16, 16)]
            w16 = plsc.load_gather(lut_v, [lax.shift_right_logical(d16, 1)])
            shmt = (d16 & 1) * 16
            half_buf[pl.ds(g * 16, 16)] = (
                lax.shift_right_logical(w16, shmt) & NOT_TGT)

        def grp(g, cur):
            half = half_buf[pl.ds(g * 16, 16)]
            m = half != NOT_TGT

            def hit(cur):
                cur = lax.cond(cur > K - 16, flush, lambda c: c, cur)
                s16 = srcc_v[pl.ds(g * 16, 16)]
                plsc.store_compressed(sbuf_v.at[pl.ds(cur, 16)], s16, mask=m)
                plsc.store_compressed(slbuf_v.at[pl.ds(cur, 16)], half,
                                      mask=m)
                return cur + jnp.sum(m.astype(jnp.int32))

            return lax.cond(jnp.any(m), hit, lambda c: c, cur)

        return lax.fori_loop(0, NGROUP, grp, cur0)

    lax.fori_loop(0, NCHUNK, chunk_body, 0)
    flush(0)

    plsc.subcore_barrier()

    @pl.when(sid == 0)
    def _():
        pltpu.sync_copy(acc_sh, acc_out.at[cid])
        pltpu.sync_copy(cnt_sh, cnt_out.at[cid])

    tbase = wid * TGT_PER_TILE
    pltpu.sync_copy(tgt_hbm.at[pl.ds(tbase, TGT_PER_TILE)], ti_v)
    pltpu.sync_copy(y_hbm.at[ti_v], tr_v)
    pltpu.sync_copy(tr_v, yt_out.at[pl.ds(tbase, TGT_PER_TILE)])


@functools.cache
def _make_sc_aggregate():
    return pl.kernel(
        _sc_body,
        out_type=(
            jax.ShapeDtypeStruct((NC, ACC_ROWS, HID), jnp.float32),
            jax.ShapeDtypeStruct((NC, ACC_ROWS, 16), jnp.float32),
            jax.ShapeDtypeStruct((NT, HID), jnp.float32),
        ),
        mesh=plsc.VectorSubcoreMesh(core_axis_name="c", subcore_axis_name="s"),
        compiler_params=pltpu.CompilerParams(needs_layout_passes=False,
                                             use_tc_tiling_on_sc=False),
        scratch_types=[
            pltpu.VMEM((N_NODES // 2,), jnp.int32),
            pltpu.VMEM((CHUNK,), jnp.int32),
            pltpu.VMEM((CHUNK,), jnp.int32),
            pltpu.VMEM((CHUNK,), jnp.int32),
            pltpu.VMEM((K,), jnp.int32),
            pltpu.VMEM((K,), jnp.int32),
            pltpu.VMEM((K, HID), jnp.float32),
            pltpu.VMEM((K, 16), jnp.float32),
            pltpu.VMEM((TGT_PER_TILE,), jnp.int32),
            pltpu.VMEM((TGT_PER_TILE, HID), jnp.float32),
            pltpu.VMEM_SHARED((ACC_ROWS, HID), jnp.float32),
            pltpu.VMEM_SHARED((ACC_ROWS, 16), jnp.float32),
        ],
    )


def _head_body(acc_ref, cnt_ref, sm_ref, yt_ref, wh_ref, bh_ref, bc_ref,
               o_ref):
    acc = acc_ref[0] + acc_ref[1]
    cntc = cnt_ref[0] + cnt_ref[1]
    cnt = cntc[:, 0:1]
    sm = sm_ref[...]
    iota_s = lax.broadcasted_iota(jnp.int32, (NT, ACC_ROWS), 1)
    p = (iota_s == sm).astype(jnp.float32)
    acc_t = jnp.dot(p, acc, preferred_element_type=jnp.float32)
    cnt_t = jnp.dot(p, cnt, preferred_element_type=jnp.float32)
    mean_t = acc_t * ALPHA / jnp.maximum(cnt_t, 1.0)
    h = jax.nn.relu(mean_t + yt_ref[...] + bc_ref[...])
    z = jnp.dot(h, wh_ref[...], preferred_element_type=jnp.float32)
    z = z + bh_ref[...]
    col = lax.broadcasted_iota(jnp.int32, (NT, 8), 1)
    o_ref[...] = jnp.where(col < 3, jax.nn.sigmoid(z), z)


def _heads(acc, cnt, slot_map, yt, wh, bh, bc):
    return pl.pallas_call(
        _head_body,
        out_shape=jax.ShapeDtypeStruct((NT, 8), jnp.float32),
    )(acc, cnt, slot_map, yt, wh, bh, bc)


@jax.jit
def kernel(x, edge_index, target_node_index, W_conv, b_conv, W_mu, b_mu,
           W_c, b_c):
    src = edge_index[0].astype(jnp.int32)
    dst = edge_index[1].astype(jnp.int32)
    tgt = target_node_index.astype(jnp.int32)

    y = _project(x, W_conv.T)

    lut = jnp.full((N_NODES,), -1, jnp.int32)
    lut = lut.at[tgt].set(jnp.arange(NT, dtype=jnp.int32))
    slot_map = lut[tgt].reshape(NT, 1)
    lutu = jnp.where(lut < 0, NOT_TGT, lut).astype(jnp.uint32)
    lut_packed = lax.bitcast_convert_type(
        lutu[0::2] | (lutu[1::2] << jnp.uint32(16)), jnp.int32)

    z64 = jnp.zeros((ACC_ROWS, HID), jnp.float32)
    z16 = jnp.zeros((ACC_ROWS, 16), jnp.float32)
    ones = jnp.ones((K, 16), jnp.float32)

    acc, cnt, yt = _make_sc_aggregate()(src, dst, lut_packed, y, tgt, z64,
                                        z16, ones)

    wh = jnp.zeros((8, HID), jnp.float32)
    wh = wh.at[0:3].set(W_mu).at[3].set(W_c[0])
    bh = jnp.zeros((1, 8), jnp.float32)
    bh = bh.at[0, 0:3].set(b_mu).at[0, 3].set(b_c[0])

    o = _heads(acc, cnt, slot_map, yt, wh.T, bh, b_conv.reshape(1, HID))

    mu = o[:, 0:3]
    state_value = o[:, 3:4]
    std = jnp.asarray(1e-05, dtype=jnp.float32)
    return (mu, std, state_value)


# speedup vs baseline: 1.1852x; 1.1156x over previous
import functools

import jax
import jax.numpy as jnp
from jax import lax
from jax.experimental import pallas as pl
from jax.experimental.pallas import tpu as pltpu
from jax.experimental.pallas import tpu_sc as plsc

N_NODES = 100000
IN_CH = 768
HID = 64
NT = 1024
N_EDGES = 3200000
ALPHA = 0.0001

NC = 2
NS = 16
NW = NC * NS
EDGES_PER_TILE = N_EDGES // NW
CHUNK = 10000
NGROUP = CHUNK // 16
NCHUNK = EDGES_PER_TILE // CHUNK
K = 128
TRASH = NT
ACC_ROWS = NT + 8
TGT_PER_TILE = NT // NW
ROW_BLK = 2000
NOT_TGT = 0xFFFF


def _proj_body(x_ref, w_ref, y_ref):
    y_ref[...] = jnp.dot(x_ref[...], w_ref[...],
                         preferred_element_type=jnp.float32)


def _project(x, wt):
    return pl.pallas_call(
        _proj_body,
        grid=(N_NODES // ROW_BLK,),
        in_specs=[
            pl.BlockSpec((ROW_BLK, IN_CH), lambda i: (i, 0)),
            pl.BlockSpec((IN_CH, HID), lambda i: (0, 0)),
        ],
        out_specs=pl.BlockSpec((ROW_BLK, HID), lambda i: (i, 0)),
        out_shape=jax.ShapeDtypeStruct((N_NODES, HID), jnp.float32),
    )(x, wt)


def _sc_body(src_hbm, dst_hbm, lut_hbm, y_hbm, tgt_hbm, z64_hbm, z16_hbm,
             ones_hbm, acc_out, cnt_out, yt_out,
             lut_v, srcc_v, dstc_v, half_buf, sbuf_v, slbuf_v, rows_v,
             ones_v, ti_v, tr_v, sem0, sem1, sem2, sem3, acc_sh, cnt_sh):
    cid = lax.axis_index("c")
    sid = lax.axis_index("s")
    wid = sid * NC + cid

    @pl.when(sid == 0)
    def _():
        pltpu.sync_copy(z64_hbm, acc_sh)
        pltpu.sync_copy(z16_hbm, cnt_sh)
    plsc.subcore_barrier()

    pltpu.sync_copy(lut_hbm, lut_v)
    pltpu.sync_copy(ones_hbm, ones_v)

    zeros16 = jnp.zeros((16,), jnp.int32)
    trash16 = jnp.full((16,), TRASH, jnp.int32)
    for i in range(K // 16):
        sbuf_v[pl.ds(16 * i, 16)] = zeros16
        slbuf_v[pl.ds(16 * i, 16)] = trash16

    def flush(_):
        pltpu.sync_copy(y_hbm.at[sbuf_v], rows_v)
        pltpu.sync_copy(rows_v, acc_sh.at[slbuf_v], add=True)
        pltpu.sync_copy(ones_v, cnt_sh.at[slbuf_v], add=True)
        for i in range(K // 16):
            slbuf_v[pl.ds(16 * i, 16)] = trash16
        return 0

    ebase = wid * EDGES_PER_TILE

    def start_load(c, src_buf, dst_buf, sem_a, sem_b):
        base = ebase + c * CHUNK
        pltpu.make_async_copy(src_hbm.at[pl.ds(base, CHUNK)], src_buf,
                              sem_a).start()
        pltpu.make_async_copy(dst_hbm.at[pl.ds(base, CHUNK)], dst_buf,
                              sem_b).start()

    def wait_load(src_buf, dst_buf, sem_a, sem_b):
        pltpu.make_async_copy(src_hbm.at[pl.ds(ebase, CHUNK)], src_buf,
                              sem_a).wait()
        pltpu.make_async_copy(dst_hbm.at[pl.ds(ebase, CHUNK)], dst_buf,
                              sem_b).wait()

    def process(src_buf, dst_buf, cur0):
        @plsc.parallel_loop(0, NGROUP, unroll=4)
        def _(g):
            d16 = dst_buf[pl.ds(g * 16, 16)]
            w16 = plsc.load_gather(lut_v, [lax.shift_right_logical(d16, 1)])
            shmt = (d16 & 1) * 16
            half_buf[pl.ds(g * 16, 16)] = (
                lax.shift_right_logical(w16, shmt) & NOT_TGT)

        def grp(g, cur):
            half = half_buf[pl.ds(g * 16, 16)]
            m = half != NOT_TGT

            def hit(cur):
                cur = lax.cond(cur > K - 16, flush, lambda c: c, cur)
                s16 = src_buf[pl.ds(g * 16, 16)]
                plsc.store_compressed(sbuf_v.at[pl.ds(cur, 16)], s16, mask=m)
                plsc.store_compressed(slbuf_v.at[pl.ds(cur, 16)], half,
                                      mask=m)
                return cur + jnp.sum(m.astype(jnp.int32))

            return lax.cond(jnp.any(m), hit, lambda c: c, cur)

        return lax.fori_loop(0, NGROUP, grp, cur0)

    start_load(0, srcc_v.at[0], dstc_v.at[0], sem0, sem1)

    def pair_body(h, cur):
        c0 = 2 * h
        wait_load(srcc_v.at[0], dstc_v.at[0], sem0, sem1)
        start_load(c0 + 1, srcc_v.at[1], dstc_v.at[1], sem2, sem3)
        cur = process(srcc_v.at[0], dstc_v.at[0], cur)
        wait_load(srcc_v.at[1], dstc_v.at[1], sem2, sem3)

        @pl.when(c0 + 2 < NCHUNK)
        def _():
            start_load(c0 + 2, srcc_v.at[0], dstc_v.at[0], sem0, sem1)
        return process(srcc_v.at[1], dstc_v.at[1], cur)

    lax.fori_loop(0, NCHUNK // 2, pair_body, 0)
    flush(0)

    plsc.subcore_barrier()

    @pl.when(sid == 0)
    def _():
        pltpu.sync_copy(acc_sh, acc_out.at[cid])
        pltpu.sync_copy(cnt_sh, cnt_out.at[cid])

    tbase = wid * TGT_PER_TILE
    pltpu.sync_copy(tgt_hbm.at[pl.ds(tbase, TGT_PER_TILE)], ti_v)
    pltpu.sync_copy(y_hbm.at[ti_v], tr_v)
    pltpu.sync_copy(tr_v, yt_out.at[pl.ds(tbase, TGT_PER_TILE)])


@functools.cache
def _make_sc_aggregate():
    return pl.kernel(
        _sc_body,
        out_type=(
            jax.ShapeDtypeStruct((NC, ACC_ROWS, HID), jnp.float32),
            jax.ShapeDtypeStruct((NC, ACC_ROWS, 16), jnp.float32),
            jax.ShapeDtypeStruct((NT, HID), jnp.float32),
        ),
        mesh=plsc.VectorSubcoreMesh(core_axis_name="c", subcore_axis_name="s"),
        compiler_params=pltpu.CompilerParams(needs_layout_passes=False,
                                             use_tc_tiling_on_sc=False),
        scratch_types=[
            pltpu.VMEM((N_NODES // 2,), jnp.int32),
            pltpu.VMEM((2, CHUNK), jnp.int32),
            pltpu.VMEM((2, CHUNK), jnp.int32),
            pltpu.VMEM((CHUNK,), jnp.int32),
            pltpu.VMEM((K,), jnp.int32),
            pltpu.VMEM((K,), jnp.int32),
            pltpu.VMEM((K, HID), jnp.float32),
            pltpu.VMEM((K, 16), jnp.float32),
            pltpu.VMEM((TGT_PER_TILE,), jnp.int32),
            pltpu.VMEM((TGT_PER_TILE, HID), jnp.float32),
            pltpu.SemaphoreType.DMA,
            pltpu.SemaphoreType.DMA,
            pltpu.SemaphoreType.DMA,
            pltpu.SemaphoreType.DMA,
            pltpu.VMEM_SHARED((ACC_ROWS, HID), jnp.float32),
            pltpu.VMEM_SHARED((ACC_ROWS, 16), jnp.float32),
        ],
    )


def _head_body(acc_ref, cnt_ref, sm_ref, yt_ref, wh_ref, bh_ref, bc_ref,
               o_ref):
    acc = acc_ref[0] + acc_ref[1]
    cntc = cnt_ref[0] + cnt_ref[1]
    cnt = cntc[:, 0:1]
    sm = sm_ref[...]
    iota_s = lax.broadcasted_iota(jnp.int32, (NT, ACC_ROWS), 1)
    p = (iota_s == sm).astype(jnp.float32)
    acc_t = jnp.dot(p, acc, preferred_element_type=jnp.float32)
    cnt_t = jnp.dot(p, cnt, preferred_element_type=jnp.float32)
    mean_t = acc_t * ALPHA / jnp.maximum(cnt_t, 1.0)
    h = jax.nn.relu(mean_t + yt_ref[...] + bc_ref[...])
    z = jnp.dot(h, wh_ref[...], preferred_element_type=jnp.float32)
    z = z + bh_ref[...]
    col = lax.broadcasted_iota(jnp.int32, (NT, 8), 1)
    o_ref[...] = jnp.where(col < 3, jax.nn.sigmoid(z), z)


def _heads(acc, cnt, slot_map, yt, wh, bh, bc):
    return pl.pallas_call(
        _head_body,
        out_shape=jax.ShapeDtypeStruct((NT, 8), jnp.float32),
    )(acc, cnt, slot_map, yt, wh, bh, bc)


@jax.jit
def kernel(x, edge_index, target_node_index, W_conv, b_conv, W_mu, b_mu,
           W_c, b_c):
    src = edge_index[0].astype(jnp.int32)
    dst = edge_index[1].astype(jnp.int32)
    tgt = target_node_index.astype(jnp.int32)

    y = _project(x, W_conv.T)

    lut = jnp.full((N_NODES,), -1, jnp.int32)
    lut = lut.at[tgt].set(jnp.arange(NT, dtype=jnp.int32))
    slot_map = lut[tgt].reshape(NT, 1)
    lutu = jnp.where(lut < 0, NOT_TGT, lut).astype(jnp.uint32)
    lut_packed = lax.bitcast_convert_type(
        lutu[0::2] | (lutu[1::2] << jnp.uint32(16)), jnp.int32)

    z64 = jnp.zeros((ACC_ROWS, HID), jnp.float32)
    z16 = jnp.zeros((ACC_ROWS, 16), jnp.float32)
    ones = jnp.ones((K, 16), jnp.float32)

    acc, cnt, yt = _make_sc_aggregate()(src, dst, lut_packed, y, tgt, z64,
                                        z16, ones)

    wh = jnp.zeros((8, HID), jnp.float32)
    wh = wh.at[0:3].set(W_mu).at[3].set(W_c[0])
    bh = jnp.zeros((1, 8), jnp.float32)
    bh = bh.at[0, 0:3].set(b_mu).at[0, 3].set(b_c[0])

    o = _heads(acc, cnt, slot_map, yt, wh.T, bh, b_conv.reshape(1, HID))

    mu = o[:, 0:3]
    state_value = o[:, 3:4]
    std = jnp.asarray(1e-05, dtype=jnp.float32)
    return (mu, std, state_value)


# speedup vs baseline: 1.4295x; 1.2061x over previous
import functools

import jax
import jax.numpy as jnp
from jax import lax
from jax.experimental import pallas as pl
from jax.experimental.pallas import tpu as pltpu
from jax.experimental.pallas import tpu_sc as plsc

N_NODES = 100000
IN_CH = 768
HID = 64
NT = 1024
N_EDGES = 3200000
ALPHA = 0.0001

NC = 2
NS = 16
NW = NC * NS
EDGES_PER_TILE = N_EDGES // NW
CHUNK = 10000
NGROUP = CHUNK // 16
NCHUNK = EDGES_PER_TILE // CHUNK
K = 128
TRASH = NT
ACC_ROWS = NT + 8
TGT_PER_TILE = NT // NW
ROW_BLK = 2000
NOT_TGT = 0xFFFF


def _proj_body(x_ref, w_ref, y_ref):
    y_ref[...] = jnp.dot(x_ref[...], w_ref[...],
                         preferred_element_type=jnp.float32)


def _project(x, wt):
    return pl.pallas_call(
        _proj_body,
        grid=(N_NODES // ROW_BLK,),
        in_specs=[
            pl.BlockSpec((ROW_BLK, IN_CH), lambda i: (i, 0)),
            pl.BlockSpec((IN_CH, HID), lambda i: (0, 0)),
        ],
        out_specs=pl.BlockSpec((ROW_BLK, HID), lambda i: (i, 0)),
        out_shape=jax.ShapeDtypeStruct((N_NODES, HID), jnp.float32),
    )(x, wt)


def _sc_body(src_hbm, dst_hbm, lut_hbm, y_hbm, tgt_hbm, z64_hbm, z16_hbm,
             ones_hbm, acc_out, cnt_out, yt_out,
             lut_v, srcc_v, dstc_v, half_buf, sbuf_v, slbuf_v, rows_v,
             ones_v, ti_v, tr_v, sem0, sem1, sem2, sem3, acc_sh, cnt_sh):
    cid = lax.axis_index("c")
    sid = lax.axis_index("s")
    wid = sid * NC + cid

    @pl.when(sid == 0)
    def _():
        pltpu.sync_copy(z64_hbm, acc_sh)
        pltpu.sync_copy(z16_hbm, cnt_sh)
    plsc.subcore_barrier()

    pltpu.sync_copy(lut_hbm, lut_v)
    pltpu.sync_copy(ones_hbm, ones_v)

    zeros16 = jnp.zeros((16,), jnp.int32)
    trash16 = jnp.full((16,), TRASH, jnp.int32)
    for i in range(K // 16):
        sbuf_v[pl.ds(16 * i, 16)] = zeros16
        slbuf_v[pl.ds(16 * i, 16)] = trash16

    def flush(_):
        pltpu.sync_copy(y_hbm.at[sbuf_v], rows_v)
        pltpu.sync_copy(rows_v, acc_sh.at[slbuf_v], add=True)
        pltpu.sync_copy(ones_v, cnt_sh.at[slbuf_v], add=True)
        for i in range(K // 16):
            slbuf_v[pl.ds(16 * i, 16)] = trash16
        return 0

    ebase = wid * EDGES_PER_TILE

    def start_load(c, src_buf, dst_buf, sem_a, sem_b):
        base = ebase + c * CHUNK
        pltpu.make_async_copy(src_hbm.at[pl.ds(base, CHUNK)], src_buf,
                              sem_a).start()
        pltpu.make_async_copy(dst_hbm.at[pl.ds(base, CHUNK)], dst_buf,
                              sem_b).start()

    def wait_load(src_buf, dst_buf, sem_a, sem_b):
        pltpu.make_async_copy(src_hbm.at[pl.ds(ebase, CHUNK)], src_buf,
                              sem_a).wait()
        pltpu.make_async_copy(dst_hbm.at[pl.ds(ebase, CHUNK)], dst_buf,
                              sem_b).wait()

    def process(src_buf, dst_buf, cur0):
        @plsc.parallel_loop(0, NGROUP, unroll=8)
        def _(g):
            d16 = dst_buf[pl.ds(g * 16, 16)]
            w16 = plsc.load_gather(lut_v, [lax.shift_right_logical(d16, 1)])
            shmt = (d16 & 1) * 16
            half_buf[pl.ds(g * 16, 16)] = (
                lax.shift_right_logical(w16, shmt) & NOT_TGT)

        def grp(g2, cur):
            half_a = half_buf[pl.ds(g2 * 32, 16)]
            half_b = half_buf[pl.ds(g2 * 32 + 16, 16)]
            m_a = half_a != NOT_TGT
            m_b = half_b != NOT_TGT

            def hit(cur):
                def emit(off, half, m, cur):
                    def go(cur):
                        cur = lax.cond(cur > K - 16, flush, lambda c: c, cur)
                        s16 = src_buf[pl.ds(g2 * 32 + off, 16)]
                        plsc.store_compressed(sbuf_v.at[pl.ds(cur, 16)],
                                              s16, mask=m)
                        plsc.store_compressed(slbuf_v.at[pl.ds(cur, 16)],
                                              half, mask=m)
                        return cur + jnp.sum(m.astype(jnp.int32))
                    return lax.cond(jnp.any(m), go, lambda c: c, cur)
                cur = emit(0, half_a, m_a, cur)
                return emit(16, half_b, m_b, cur)

            return lax.cond(jnp.any(m_a | m_b), hit, lambda c: c, cur)

        return lax.fori_loop(0, NGROUP // 2, grp, cur0)

    start_load(0, srcc_v.at[0], dstc_v.at[0], sem0, sem1)

    def pair_body(h, cur):
        c0 = 2 * h
        wait_load(srcc_v.at[0], dstc_v.at[0], sem0, sem1)
        start_load(c0 + 1, srcc_v.at[1], dstc_v.at[1], sem2, sem3)
        cur = process(srcc_v.at[0], dstc_v.at[0], cur)
        wait_load(srcc_v.at[1], dstc_v.at[1], sem2, sem3)

        @pl.when(c0 + 2 < NCHUNK)
        def _():
            start_load(c0 + 2, srcc_v.at[0], dstc_v.at[0], sem0, sem1)
        return process(srcc_v.at[1], dstc_v.at[1], cur)

    lax.fori_loop(0, NCHUNK // 2, pair_body, 0)
    flush(0)

    plsc.subcore_barrier()

    @pl.when(sid == 0)
    def _():
        pltpu.sync_copy(acc_sh, acc_out.at[cid])
        pltpu.sync_copy(cnt_sh, cnt_out.at[cid])

    tbase = wid * TGT_PER_TILE
    pltpu.sync_copy(tgt_hbm.at[pl.ds(tbase, TGT_PER_TILE)], ti_v)
    pltpu.sync_copy(y_hbm.at[ti_v], tr_v)
    pltpu.sync_copy(tr_v, yt_out.at[pl.ds(tbase, TGT_PER_TILE)])


@functools.cache
def _make_sc_aggregate():
    return pl.kernel(
        _sc_body,
        out_type=(
            jax.ShapeDtypeStruct((NC, ACC_ROWS, HID), jnp.float32),
            jax.ShapeDtypeStruct((NC, ACC_ROWS, 16), jnp.float32),
            jax.ShapeDtypeStruct((NT, HID), jnp.float32),
        ),
        mesh=plsc.VectorSubcoreMesh(core_axis_name="c", subcore_axis_name="s"),
        compiler_params=pltpu.CompilerParams(needs_layout_passes=False,
                                             use_tc_tiling_on_sc=False),
        scratch_types=[
            pltpu.VMEM((N_NODES // 2,), jnp.int32),
            pltpu.VMEM((2, CHUNK), jnp.int32),
            pltpu.VMEM((2, CHUNK), jnp.int32),
            pltpu.VMEM((CHUNK,), jnp.int32),
            pltpu.VMEM((K,), jnp.int32),
            pltpu.VMEM((K,), jnp.int32),
            pltpu.VMEM((K, HID), jnp.float32),
            pltpu.VMEM((K, 16), jnp.float32),
            pltpu.VMEM((TGT_PER_TILE,), jnp.int32),
            pltpu.VMEM((TGT_PER_TILE, HID), jnp.float32),
            pltpu.SemaphoreType.DMA,
            pltpu.SemaphoreType.DMA,
            pltpu.SemaphoreType.DMA,
            pltpu.SemaphoreType.DMA,
            pltpu.VMEM_SHARED((ACC_ROWS, HID), jnp.float32),
            pltpu.VMEM_SHARED((ACC_ROWS, 16), jnp.float32),
        ],
    )


def _head_body(acc_ref, cnt_ref, sm_ref, yt_ref, wh_ref, bh_ref, bc_ref,
               o_ref):
    acc = acc_ref[0] + acc_ref[1]
    cntc = cnt_ref[0] + cnt_ref[1]
    cnt = cntc[:, 0:1]
    sm = sm_ref[...]
    iota_s = lax.broadcasted_iota(jnp.int32, (NT, ACC_ROWS), 1)
    p = (iota_s == sm).astype(jnp.float32)
    acc_t = jnp.dot(p, acc, preferred_element_type=jnp.float32)
    cnt_t = jnp.dot(p, cnt, preferred_element_type=jnp.float32)
    mean_t = acc_t * ALPHA / jnp.maximum(cnt_t, 1.0)
    h = jax.nn.relu(mean_t + yt_ref[...] + bc_ref[...])
    z = jnp.dot(h, wh_ref[...], preferred_element_type=jnp.float32)
    z = z + bh_ref[...]
    col = lax.broadcasted_iota(jnp.int32, (NT, 8), 1)
    o_ref[...] = jnp.where(col < 3, jax.nn.sigmoid(z), z)


def _heads(acc, cnt, slot_map, yt, wh, bh, bc):
    return pl.pallas_call(
        _head_body,
        out_shape=jax.ShapeDtypeStruct((NT, 8), jnp.float32),
    )(acc, cnt, slot_map, yt, wh, bh, bc)


@jax.jit
def kernel(x, edge_index, target_node_index, W_conv, b_conv, W_mu, b_mu,
           W_c, b_c):
    src = edge_index[0].astype(jnp.int32)
    dst = edge_index[1].astype(jnp.int32)
    tgt = target_node_index.astype(jnp.int32)

    y = _project(x, W_conv.T)

    lut = jnp.full((N_NODES,), -1, jnp.int32)
    lut = lut.at[tgt].set(jnp.arange(NT, dtype=jnp.int32))
    slot_map = lut[tgt].reshape(NT, 1)
    lutu = jnp.where(lut < 0, NOT_TGT, lut).astype(jnp.uint32)
    lut_packed = lax.bitcast_convert_type(
        lutu[0::2] | (lutu[1::2] << jnp.uint32(16)), jnp.int32)

    z64 = jnp.zeros((ACC_ROWS, HID), jnp.float32)
    z16 = jnp.zeros((ACC_ROWS, 16), jnp.float32)
    ones = jnp.ones((K, 16), jnp.float32)

    acc, cnt, yt = _make_sc_aggregate()(src, dst, lut_packed, y, tgt, z64,
                                        z16, ones)

    wh = jnp.zeros((8, HID), jnp.float32)
    wh = wh.at[0:3].set(W_mu).at[3].set(W_c[0])
    bh = jnp.zeros((1, 8), jnp.float32)
    bh = bh.at[0, 0:3].set(b_mu).at[0, 3].set(b_c[0])

    o = _heads(acc, cnt, slot_map, yt, wh.T, bh, b_conv.reshape(1, HID))

    mu = o[:, 0:3]
    state_value = o[:, 3:4]
    std = jnp.asarray(1e-05, dtype=jnp.float32)
    return (mu, std, state_value)
